# baseline (speedup 1.0000x reference)
import functools

import jax
import jax.numpy as jnp
from jax import lax
from jax.experimental import pallas as pl
from jax.experimental.pallas import tpu as pltpu
from jax.experimental.pallas import tpu_sc as plsc

_B, _T, _N = 1024, 4, 64
_OBS_F, _D, _A = 32, 64, 16
_E = 64
_BN = _B * _N

_NC, _NS = 2, 16
_NW = _NC * _NS
_ROWS_W = _BN // _NW
_CH = 128
_NCHUNK = _ROWS_W // _CH

_BB = 32


_P = 4 * _D


def _sc_scatter_body(idx_hbm, p_hbm, y_hbm, idx_v, rows_v, sem):
    wid = lax.axis_index("s") * _NC + lax.axis_index("c")
    base = wid * _ROWS_W
    pltpu.sync_copy(idx_hbm.at[pl.ds(wid * _NCHUNK, _NCHUNK)], idx_v)

    def chunk(j, carry):
        src = base + j * _CH
        pltpu.sync_copy(p_hbm.at[pl.ds(src, _CH)], rows_v)
        pltpu.async_copy(rows_v, y_hbm.at[idx_v.at[j]], sem).wait()
        return carry

    lax.fori_loop(0, _NCHUNK, chunk, 0)


def _sc_gather_body(idx_hbm, y_hbm, g_hbm, idx_v, rows_v, sem):
    wid = lax.axis_index("s") * _NC + lax.axis_index("c")
    base = wid * _ROWS_W
    pltpu.sync_copy(idx_hbm.at[pl.ds(wid * _NCHUNK, _NCHUNK)], idx_v)

    def chunk(j, carry):
        dst = base + j * _CH
        pltpu.async_copy(y_hbm.at[idx_v.at[j]], rows_v, sem).wait()
        pltpu.sync_copy(rows_v, g_hbm.at[pl.ds(dst, _CH)])
        return carry

    lax.fori_loop(0, _NCHUNK, chunk, 0)


@functools.lru_cache(maxsize=None)
def _make_sc_kernels():
    mesh = plsc.VectorSubcoreMesh(core_axis_name="c", subcore_axis_name="s")
    kw = dict(
        mesh=mesh,
        out_type=jax.ShapeDtypeStruct((_BN, _P), jnp.float32),
        scratch_types=[
            pltpu.VMEM((_NCHUNK, _CH), jnp.int32),
            pltpu.VMEM((_CH, _P), jnp.float32),
            pltpu.SemaphoreType.DMA,
        ],
    )
    return (pl.kernel(_sc_scatter_body, **kw),
            pl.kernel(_sc_gather_body, **kw))


def _tc_body(obs_ref, act_ref, gm_ref,
             wgd_ref, wg_o_ref, bg_ref, wio_ref, who_ref, bo_ref,
             wie_ref, whe_ref, be_ref,
             oseq_ref, of_ref, hf_ref, cf_ref):
    bb = gm_ref.shape[0]
    bf = lambda x: x.astype(jnp.bfloat16)
    mm = lambda a, w: jnp.dot(bf(a), w, preferred_element_type=jnp.float32)
    sg = lambda x: 0.5 + 0.5 * jnp.tanh(0.5 * x)
    gm = gm_ref[...]
    o = gm[..., 0:_D]
    h = gm[..., _D:2 * _D]
    c = gm[..., 2 * _D:3 * _D]
    wg_o = bf(wg_o_ref[...])
    bg = bg_ref[...]
    wio = bf(wio_ref[...])
    who = bf(who_ref[...])
    bo = bo_ref[...]
    wie_e = bf(wie_ref[0:_E, :])
    wie_a = bf(wie_ref[_E:, :])
    whe = bf(whe_ref[...])
    be = be_ref[...]
    obs2 = obs_ref[...].reshape(bb * _N, _T * _OBS_F)
    epre = mm(obs2, bf(wgd_ref[...]))
    for t in range(_T):
        o2 = o.reshape(bb * _N, _D)
        h2 = h.reshape(bb * _N, _D)
        c2 = c.reshape(bb * _N, _D)
        e = jnp.tanh(epre[:, t * _E:(t + 1) * _E] + mm(o2, wg_o) + bg)
        g = mm(e, wio) + mm(h2, who) + bo
        ii = sg(g[:, 0:_D])
        ff = sg(g[:, _D:2 * _D])
        gg = jnp.tanh(g[:, 2 * _D:3 * _D])
        og = sg(g[:, 3 * _D:4 * _D])
        cn = ff * c2 + ii * gg
        hn = og * jnp.tanh(cn)
        e0 = e.reshape(bb, _N, _E)[:, 0, :]
        a_t = act_ref[t]
        h0 = h[:, 0, :]
        c0 = c[:, 0, :]
        ge = mm(e0, wie_e) + mm(a_t, wie_a) + mm(h0, whe) + be
        ie = sg(ge[:, 0:_D])
        fe = sg(ge[:, _D:2 * _D])
        gge = jnp.tanh(ge[:, 2 * _D:3 * _D])
        oe = sg(ge[:, 3 * _D:4 * _D])
        ce = fe * c0 + ie * gge
        he = oe * jnp.tanh(ce)
        n_iota = lax.broadcasted_iota(jnp.int32, (bb, _N, _D), 1)
        h = jnp.where(n_iota == 0, he[:, None, :],
                      hn.reshape(bb, _N, _D))
        c = jnp.where(n_iota == 0, ce[:, None, :],
                      cn.reshape(bb, _N, _D))
        o = h
        oseq_ref[:, t] = h
    of_ref[...] = h
    hf_ref[...] = h
    cf_ref[...] = c


def _tc_call(obs_p, act_t, gm, W_gd, Wg_o, b_gnn, W_ih_oth,
             W_hh_oth, b_oth, W_ih_ego, W_hh_ego, b_ego, interpret=False):
    nblk = _B // _BB
    full = lambda s: pl.BlockSpec(s, lambda i: tuple(0 for _ in s))
    grid_spec = pl.GridSpec(
        grid=(nblk,),
        in_specs=[
            pl.BlockSpec((_BB, _N, _T * _OBS_F), lambda i: (i, 0, 0)),
            pl.BlockSpec((_T, _BB, _A), lambda i: (0, i, 0)),
            pl.BlockSpec((_BB, _N, _P), lambda i: (i, 0, 0)),
            full((_T * _OBS_F, _T * _E)),
            full((_D, _E)),
            full((1, _E)),
            full((_E, 4 * _D)),
            full((_D, 4 * _D)),
            full((1, 4 * _D)),
            full((_E + _A, 4 * _D)),
            full((_D, 4 * _D)),
            full((1, 4 * _D)),
        ],
        out_specs=[
            pl.BlockSpec((_BB, _T, _N, _D), lambda i: (i, 0, 0, 0)),
            pl.BlockSpec((_BB, _N, _D), lambda i: (i, 0, 0)),
            pl.BlockSpec((_BB, _N, _D), lambda i: (i, 0, 0)),
            pl.BlockSpec((_BB, _N, _D), lambda i: (i, 0, 0)),
        ],
    )
    return pl.pallas_call(
        _tc_body,
        grid_spec=grid_spec,
        out_shape=[
            jax.ShapeDtypeStruct((_B, _T, _N, _D), jnp.float32),
            jax.ShapeDtypeStruct((_B, _N, _D), jnp.float32),
            jax.ShapeDtypeStruct((_B, _N, _D), jnp.float32),
            jax.ShapeDtypeStruct((_B, _N, _D), jnp.float32),
        ],
        interpret=interpret,
    )(obs_p, act_t, gm, W_gd, Wg_o, b_gnn.reshape(1, _E),
      W_ih_oth, W_hh_oth, b_oth.reshape(1, 4 * _D), W_ih_ego, W_hh_ego,
      b_ego.reshape(1, 4 * _D))


def kernel(obs_sequence, action_sequence, o0, h0, c0, W_gnn, b_gnn,
           W_ih_ego, W_hh_ego, b_ego, W_ih_oth, W_hh_oth, b_oth):
    obs4 = obs_sequence.reshape(_B, _T, _N, _OBS_F + 1)
    ids_last = obs4[:, _T - 1, :, _OBS_F:]
    cur_idx = obs4[:, 0, :, _OBS_F].reshape(-1).astype(jnp.int32)
    lat_idx = o0[:, :, _D].reshape(-1).astype(jnp.int32)
    of = o0[:, :, :_D].reshape(_BN, _D)
    hf = h0[:, :, :_D].reshape(_BN, _D)
    cf = c0[:, :, :_D].reshape(_BN, _D)
    packed = jnp.concatenate([of, hf, cf, cf], axis=-1)

    sc_scatter, sc_gather = _make_sc_kernels()
    y = sc_scatter(lat_idx.reshape(_NW * _NCHUNK, _CH), packed)
    g = sc_gather(cur_idx.reshape(_NW * _NCHUNK, _CH), y)

    obs_p = obs4[..., :_OBS_F].transpose(0, 2, 1, 3).reshape(
        _B, _N, _T * _OBS_F)
    wg_obs = W_gnn[:_OBS_F, :]
    W_gd = jnp.zeros((_T * _OBS_F, _T * _E), jnp.float32)
    for t in range(_T):
        W_gd = W_gd.at[t * _OBS_F:(t + 1) * _OBS_F,
                       t * _E:(t + 1) * _E].set(wg_obs)
    Wg_o = W_gnn[_OBS_F:, :]

    act_t = action_sequence.transpose(1, 0, 2)
    oseq, ofin, hfin, cfin = _tc_call(
        obs_p, act_t, g.reshape(_B, _N, _P),
        W_gd, Wg_o, b_gnn, W_ih_oth, W_hh_oth, b_oth,
        W_ih_ego, W_hh_ego, b_ego)

    o_out = jnp.concatenate([ofin, ids_last], axis=-1)
    h_out = jnp.concatenate([hfin, ids_last], axis=-1)
    c_out = jnp.concatenate([cfin, ids_last], axis=-1)
    return oseq, (o_out, h_out, c_out)


# speedup vs baseline: 1.1072x; 1.1072x over previous
import functools

import jax
import jax.numpy as jnp
from jax import lax
from jax.experimental import pallas as pl
from jax.experimental.pallas import tpu as pltpu
from jax.experimental.pallas import tpu_sc as plsc

_B, _T, _N = 1024, 4, 64
_OBS_F, _D, _A = 32, 64, 16
_E = 64
_BN = _B * _N

_NC, _NS = 2, 16
_NW = _NC * _NS
_ROWS_W = _BN // _NW
_CH = 128
_NCHUNK = _ROWS_W // _CH

_BB = 64


_P = 4 * _D


def _sc_scatter_body(idx_hbm, p_hbm, y_hbm, idx_v, rows_v, sem):
    wid = lax.axis_index("s") * _NC + lax.axis_index("c")
    base = wid * _ROWS_W
    pltpu.sync_copy(idx_hbm.at[pl.ds(wid * _NCHUNK, _NCHUNK)], idx_v)

    def chunk(j, carry):
        src = base + j * _CH
        pltpu.sync_copy(p_hbm.at[pl.ds(src, _CH)], rows_v)
        pltpu.async_copy(rows_v, y_hbm.at[idx_v.at[j]], sem).wait()
        return carry

    lax.fori_loop(0, _NCHUNK, chunk, 0)


def _sc_gather_body(idx_hbm, y_hbm, g_hbm, idx_v, rows_v, sem):
    wid = lax.axis_index("s") * _NC + lax.axis_index("c")
    base = wid * _ROWS_W
    pltpu.sync_copy(idx_hbm.at[pl.ds(wid * _NCHUNK, _NCHUNK)], idx_v)

    def chunk(j, carry):
        dst = base + j * _CH
        pltpu.async_copy(y_hbm.at[idx_v.at[j]], rows_v, sem).wait()
        pltpu.sync_copy(rows_v, g_hbm.at[pl.ds(dst, _CH)])
        return carry

    lax.fori_loop(0, _NCHUNK, chunk, 0)


@functools.lru_cache(maxsize=None)
def _make_sc_kernels():
    mesh = plsc.VectorSubcoreMesh(core_axis_name="c", subcore_axis_name="s")
    kw = dict(
        mesh=mesh,
        out_type=jax.ShapeDtypeStruct((_BN, _P), jnp.float32),
        scratch_types=[
            pltpu.VMEM((_NCHUNK, _CH), jnp.int32),
            pltpu.VMEM((_CH, _P), jnp.float32),
            pltpu.SemaphoreType.DMA,
        ],
    )
    return (pl.kernel(_sc_scatter_body, **kw),
            pl.kernel(_sc_gather_body, **kw))


def _tc_body(obs_ref, act_ref, ids_ref, gm_ref,
             wgd_ref, wg_o_ref, bg_ref, wio_ref, who_ref, bo_ref,
             wie_ref, whe_ref, be_ref,
             oseq_ref, of_ref, hf_ref, cf_ref):
    bb = gm_ref.shape[0]
    bf = lambda x: x.astype(jnp.bfloat16)
    mm = lambda a, w: jnp.dot(bf(a), w, preferred_element_type=jnp.float32)
    sg = lambda x: 0.5 + 0.5 * jnp.tanh(0.5 * x)
    gm = gm_ref[...]
    o = gm[..., 0:_D]
    h = gm[..., _D:2 * _D]
    c = gm[..., 2 * _D:3 * _D]
    wg_o = bf(wg_o_ref[...])
    bg = bg_ref[...]
    wio = bf(wio_ref[...])
    who = bf(who_ref[...])
    bo = bo_ref[...]
    wie_e = bf(wie_ref[0:_E, :])
    wie_a = bf(wie_ref[_E:, :])
    whe = bf(whe_ref[...])
    be = be_ref[...]
    obs2 = obs_ref[...].reshape(bb * _N, _T * _OBS_F)
    epre = mm(obs2, bf(wgd_ref[...]))
    for t in range(_T):
        o2 = o.reshape(bb * _N, _D)
        h2 = h.reshape(bb * _N, _D)
        c2 = c.reshape(bb * _N, _D)
        e = jnp.tanh(epre[:, t * _E:(t + 1) * _E] + mm(o2, wg_o) + bg)
        g = mm(e, wio) + mm(h2, who) + bo
        ii = sg(g[:, 0:_D])
        ff = sg(g[:, _D:2 * _D])
        gg = jnp.tanh(g[:, 2 * _D:3 * _D])
        og = sg(g[:, 3 * _D:4 * _D])
        cn = ff * c2 + ii * gg
        hn = og * jnp.tanh(cn)
        e0 = e.reshape(bb, _N, _E)[:, 0, :]
        a_t = act_ref[t]
        h0 = h[:, 0, :]
        c0 = c[:, 0, :]
        ge = mm(e0, wie_e) + mm(a_t, wie_a) + mm(h0, whe) + be
        ie = sg(ge[:, 0:_D])
        fe = sg(ge[:, _D:2 * _D])
        gge = jnp.tanh(ge[:, 2 * _D:3 * _D])
        oe = sg(ge[:, 3 * _D:4 * _D])
        ce = fe * c0 + ie * gge
        he = oe * jnp.tanh(ce)
        n_iota = lax.broadcasted_iota(jnp.int32, (bb, _N, _D), 1)
        h = jnp.where(n_iota == 0, he[:, None, :],
                      hn.reshape(bb, _N, _D))
        c = jnp.where(n_iota == 0, ce[:, None, :],
                      cn.reshape(bb, _N, _D))
        o = h
        oseq_ref[:, t] = h
    ids = ids_ref[...]
    hout = jnp.concatenate([h, ids], axis=-1)
    of_ref[...] = hout
    hf_ref[...] = hout
    cf_ref[...] = jnp.concatenate([c, ids], axis=-1)


def _tc_call(obs_p, act_t, ids_last, gm, W_gd, Wg_o, b_gnn, W_ih_oth,
             W_hh_oth, b_oth, W_ih_ego, W_hh_ego, b_ego, interpret=False):
    nblk = _B // _BB
    full = lambda s: pl.BlockSpec(s, lambda i: tuple(0 for _ in s))
    grid_spec = pl.GridSpec(
        grid=(nblk,),
        in_specs=[
            pl.BlockSpec((_BB, _N, _T * _OBS_F), lambda i: (i, 0, 0)),
            pl.BlockSpec((_T, _BB, _A), lambda i: (0, i, 0)),
            pl.BlockSpec((_BB, _N, 1), lambda i: (i, 0, 0)),
            pl.BlockSpec((_BB, _N, _P), lambda i: (i, 0, 0)),
            full((_T * _OBS_F, _T * _E)),
            full((_D, _E)),
            full((1, _E)),
            full((_E, 4 * _D)),
            full((_D, 4 * _D)),
            full((1, 4 * _D)),
            full((_E + _A, 4 * _D)),
            full((_D, 4 * _D)),
            full((1, 4 * _D)),
        ],
        out_specs=[
            pl.BlockSpec((_BB, _T, _N, _D), lambda i: (i, 0, 0, 0)),
            pl.BlockSpec((_BB, _N, _D + 1), lambda i: (i, 0, 0)),
            pl.BlockSpec((_BB, _N, _D + 1), lambda i: (i, 0, 0)),
            pl.BlockSpec((_BB, _N, _D + 1), lambda i: (i, 0, 0)),
        ],
    )
    return pl.pallas_call(
        _tc_body,
        grid_spec=grid_spec,
        out_shape=[
            jax.ShapeDtypeStruct((_B, _T, _N, _D), jnp.float32),
            jax.ShapeDtypeStruct((_B, _N, _D + 1), jnp.float32),
            jax.ShapeDtypeStruct((_B, _N, _D + 1), jnp.float32),
            jax.ShapeDtypeStruct((_B, _N, _D + 1), jnp.float32),
        ],
        interpret=interpret,
    )(obs_p, act_t, ids_last, gm, W_gd, Wg_o, b_gnn.reshape(1, _E),
      W_ih_oth, W_hh_oth, b_oth.reshape(1, 4 * _D), W_ih_ego, W_hh_ego,
      b_ego.reshape(1, 4 * _D))


def kernel(obs_sequence, action_sequence, o0, h0, c0, W_gnn, b_gnn,
           W_ih_ego, W_hh_ego, b_ego, W_ih_oth, W_hh_oth, b_oth):
    obs4 = obs_sequence.reshape(_B, _T, _N, _OBS_F + 1)
    ids_last = obs4[:, _T - 1, :, _OBS_F:]
    cur_idx = obs4[:, 0, :, _OBS_F].reshape(-1).astype(jnp.int32)
    lat_idx = o0[:, :, _D].reshape(-1).astype(jnp.int32)
    of = o0[:, :, :_D].reshape(_BN, _D)
    hf = h0[:, :, :_D].reshape(_BN, _D)
    cf = c0[:, :, :_D].reshape(_BN, _D)
    packed = jnp.concatenate([of, hf, cf, cf], axis=-1)

    sc_scatter, sc_gather = _make_sc_kernels()
    y = sc_scatter(lat_idx.reshape(_NW * _NCHUNK, _CH), packed)
    g = sc_gather(cur_idx.reshape(_NW * _NCHUNK, _CH), y)

    obs_p = obs4[..., :_OBS_F].transpose(0, 2, 1, 3).reshape(
        _B, _N, _T * _OBS_F)
    wg_obs = W_gnn[:_OBS_F, :]
    W_gd = jnp.zeros((_T * _OBS_F, _T * _E), jnp.float32)
    for t in range(_T):
        W_gd = W_gd.at[t * _OBS_F:(t + 1) * _OBS_F,
                       t * _E:(t + 1) * _E].set(wg_obs)
    Wg_o = W_gnn[_OBS_F:, :]

    act_t = action_sequence.transpose(1, 0, 2)
    oseq, o_out, h_out, c_out = _tc_call(
        obs_p, act_t, ids_last, g.reshape(_B, _N, _P),
        W_gd, Wg_o, b_gnn, W_ih_oth, W_hh_oth, b_oth,
        W_ih_ego, W_hh_ego, b_ego)

    return oseq, (o_out, h_out, c_out)


# speedup vs baseline: 1.1492x; 1.0379x over previous
import functools

import jax
import jax.numpy as jnp
from jax import lax
from jax.experimental import pallas as pl
from jax.experimental.pallas import tpu as pltpu
from jax.experimental.pallas import tpu_sc as plsc

_B, _T, _N = 1024, 4, 64
_OBS_F, _D, _A = 32, 64, 16
_E = 64
_BN = _B * _N

_NC, _NS = 2, 16
_NW = _NC * _NS
_ROWS_W = _BN // _NW
_CH = 128
_NCHUNK = _ROWS_W // _CH

_BB = 64


_P = 4 * _D


def _sc_perm_body(lat_hbm, cur_hbm, perm_hbm, inv_v, stage_v, perm_v):
    wid = lax.axis_index("s") * _NC + lax.axis_index("c")
    base = wid * _ROWS_W

    def outer(cb, carry):
        pltpu.sync_copy(lat_hbm.at[pl.ds(cb * _ROWS_W, _ROWS_W)], stage_v)

        def inner(gi, carry2):
            v = stage_v[pl.ds(gi * 16, 16)]
            vals = cb * _ROWS_W + gi * 16 + lax.iota(jnp.int32, 16)
            plsc.store_scatter(inv_v, [v], vals)
            return carry2

        lax.fori_loop(0, _ROWS_W // 16, inner, 0)
        return carry

    lax.fori_loop(0, _NW, outer, 0)

    pltpu.sync_copy(cur_hbm.at[pl.ds(base, _ROWS_W)], stage_v)

    def pgroup(gi, carry):
        idx = stage_v[pl.ds(gi * 16, 16)]
        perm_v[pl.ds(gi * 16, 16)] = plsc.load_gather(inv_v, [idx])
        return carry

    lax.fori_loop(0, _ROWS_W // 16, pgroup, 0)
    pltpu.sync_copy(perm_v, perm_hbm.at[pl.ds(base, _ROWS_W)])


def _sc_gather_body(perm_hbm, p_hbm, g_hbm, idx_v, rows_v, sem):
    wid = lax.axis_index("s") * _NC + lax.axis_index("c")
    base = wid * _ROWS_W
    pltpu.sync_copy(perm_hbm.at[pl.ds(base, _ROWS_W)], idx_v)

    def chunk(j, carry):
        pltpu.async_copy(p_hbm.at[idx_v.at[pl.ds(j * _CH, _CH)]],
                         rows_v, sem).wait()
        pltpu.sync_copy(rows_v, g_hbm.at[pl.ds(base + j * _CH, _CH)])
        return carry

    lax.fori_loop(0, _NCHUNK, chunk, 0)


@functools.lru_cache(maxsize=None)
def _make_sc_kernels():
    mesh = plsc.VectorSubcoreMesh(core_axis_name="c", subcore_axis_name="s")
    perm_k = pl.kernel(
        _sc_perm_body,
        mesh=mesh,
        compiler_params=pltpu.CompilerParams(use_tc_tiling_on_sc=False,
                                             needs_layout_passes=False),
        out_type=jax.ShapeDtypeStruct((_BN,), jnp.int32),
        scratch_types=[
            pltpu.VMEM((_BN,), jnp.int32),
            pltpu.VMEM((_ROWS_W,), jnp.int32),
            pltpu.VMEM((_ROWS_W,), jnp.int32),
        ],
    )
    gather_k = pl.kernel(
        _sc_gather_body,
        mesh=mesh,
        out_type=jax.ShapeDtypeStruct((_BN, _P), jnp.float32),
        scratch_types=[
            pltpu.VMEM((_ROWS_W,), jnp.int32),
            pltpu.VMEM((_CH, _P), jnp.float32),
            pltpu.SemaphoreType.DMA,
        ],
    )
    return perm_k, gather_k


def _tc_body(obs_ref, act_ref, ids_ref, gm_ref,
             wgd_ref, wg_o_ref, bg_ref, wio_ref, who_ref, bo_ref,
             wie_ref, whe_ref, be_ref,
             oseq_ref, of_ref, hf_ref, cf_ref):
    bb = gm_ref.shape[0]
    bf = lambda x: x.astype(jnp.bfloat16)
    mm = lambda a, w: jnp.dot(bf(a), w, preferred_element_type=jnp.float32)
    sg = lambda x: 0.5 + 0.5 * jnp.tanh(0.5 * x)
    gm = gm_ref[...]
    o = gm[..., 0:_D]
    h = gm[..., _D:2 * _D]
    c = gm[..., 2 * _D:3 * _D]
    wg_o = bf(wg_o_ref[...])
    bg = bg_ref[...]
    wio = bf(wio_ref[...])
    who = bf(who_ref[...])
    bo = bo_ref[...]
    wie_e = bf(wie_ref[0:_E, :])
    wie_a = bf(wie_ref[_E:, :])
    whe = bf(whe_ref[...])
    be = be_ref[...]
    obs2 = obs_ref[...].reshape(bb * _N, _T * _OBS_F)
    epre = mm(obs2, bf(wgd_ref[...]))
    for t in range(_T):
        o2 = o.reshape(bb * _N, _D)
        h2 = h.reshape(bb * _N, _D)
        c2 = c.reshape(bb * _N, _D)
        e = jnp.tanh(epre[:, t * _E:(t + 1) * _E] + mm(o2, wg_o) + bg)
        g = mm(e, wio) + mm(h2, who) + bo
        ii = sg(g[:, 0:_D])
        ff = sg(g[:, _D:2 * _D])
        gg = jnp.tanh(g[:, 2 * _D:3 * _D])
        og = sg(g[:, 3 * _D:4 * _D])
        cn = ff * c2 + ii * gg
        hn = og * jnp.tanh(cn)
        e0 = e.reshape(bb, _N, _E)[:, 0, :]
        a_t = act_ref[t]
        h0 = h[:, 0, :]
        c0 = c[:, 0, :]
        ge = mm(e0, wie_e) + mm(a_t, wie_a) + mm(h0, whe) + be
        ie = sg(ge[:, 0:_D])
        fe = sg(ge[:, _D:2 * _D])
        gge = jnp.tanh(ge[:, 2 * _D:3 * _D])
        oe = sg(ge[:, 3 * _D:4 * _D])
        ce = fe * c0 + ie * gge
        he = oe * jnp.tanh(ce)
        n_iota = lax.broadcasted_iota(jnp.int32, (bb, _N, _D), 1)
        h = jnp.where(n_iota == 0, he[:, None, :],
                      hn.reshape(bb, _N, _D))
        c = jnp.where(n_iota == 0, ce[:, None, :],
                      cn.reshape(bb, _N, _D))
        o = h
        oseq_ref[:, t] = h
    ids = ids_ref[...]
    hout = jnp.concatenate([h, ids], axis=-1)
    of_ref[...] = hout
    hf_ref[...] = hout
    cf_ref[...] = jnp.concatenate([c, ids], axis=-1)


def _tc_call(obs_p, act_t, ids_last, gm, W_gd, Wg_o, b_gnn, W_ih_oth,
             W_hh_oth, b_oth, W_ih_ego, W_hh_ego, b_ego, interpret=False):
    nblk = _B // _BB
    full = lambda s: pl.BlockSpec(s, lambda i: tuple(0 for _ in s))
    grid_spec = pl.GridSpec(
        grid=(nblk,),
        in_specs=[
            pl.BlockSpec((_BB, _N, _T * _OBS_F), lambda i: (i, 0, 0)),
            pl.BlockSpec((_T, _BB, _A), lambda i: (0, i, 0)),
            pl.BlockSpec((_BB, _N, 1), lambda i: (i, 0, 0)),
            pl.BlockSpec((_BB, _N, _P), lambda i: (i, 0, 0)),
            full((_T * _OBS_F, _T * _E)),
            full((_D, _E)),
            full((1, _E)),
            full((_E, 4 * _D)),
            full((_D, 4 * _D)),
            full((1, 4 * _D)),
            full((_E + _A, 4 * _D)),
            full((_D, 4 * _D)),
            full((1, 4 * _D)),
        ],
        out_specs=[
            pl.BlockSpec((_BB, _T, _N, _D), lambda i: (i, 0, 0, 0)),
            pl.BlockSpec((_BB, _N, _D + 1), lambda i: (i, 0, 0)),
            pl.BlockSpec((_BB, _N, _D + 1), lambda i: (i, 0, 0)),
            pl.BlockSpec((_BB, _N, _D + 1), lambda i: (i, 0, 0)),
        ],
    )
    return pl.pallas_call(
        _tc_body,
        grid_spec=grid_spec,
        out_shape=[
            jax.ShapeDtypeStruct((_B, _T, _N, _D), jnp.float32),
            jax.ShapeDtypeStruct((_B, _N, _D + 1), jnp.float32),
            jax.ShapeDtypeStruct((_B, _N, _D + 1), jnp.float32),
            jax.ShapeDtypeStruct((_B, _N, _D + 1), jnp.float32),
        ],
        interpret=interpret,
    )(obs_p, act_t, ids_last, gm, W_gd, Wg_o, b_gnn.reshape(1, _E),
      W_ih_oth, W_hh_oth, b_oth.reshape(1, 4 * _D), W_ih_ego, W_hh_ego,
      b_ego.reshape(1, 4 * _D))


def kernel(obs_sequence, action_sequence, o0, h0, c0, W_gnn, b_gnn,
           W_ih_ego, W_hh_ego, b_ego, W_ih_oth, W_hh_oth, b_oth):
    obs4 = obs_sequence.reshape(_B, _T, _N, _OBS_F + 1)
    ids_last = obs4[:, _T - 1, :, _OBS_F:]
    cur_idx = obs4[:, 0, :, _OBS_F].reshape(-1).astype(jnp.int32)
    lat_idx = o0[:, :, _D].reshape(-1).astype(jnp.int32)
    of = o0[:, :, :_D].reshape(_BN, _D)
    hf = h0[:, :, :_D].reshape(_BN, _D)
    cf = c0[:, :, :_D].reshape(_BN, _D)
    packed = jnp.concatenate([of, hf, cf, cf], axis=-1)

    sc_perm, sc_gather = _make_sc_kernels()
    perm = sc_perm(lat_idx, cur_idx)
    g = sc_gather(perm, packed)

    obs_p = obs4[..., :_OBS_F].transpose(0, 2, 1, 3).reshape(
        _B, _N, _T * _OBS_F)
    wg_obs = W_gnn[:_OBS_F, :]
    W_gd = jnp.zeros((_T * _OBS_F, _T * _E), jnp.float32)
    for t in range(_T):
        W_gd = W_gd.at[t * _OBS_F:(t + 1) * _OBS_F,
                       t * _E:(t + 1) * _E].set(wg_obs)
    Wg_o = W_gnn[_OBS_F:, :]

    act_t = action_sequence.transpose(1, 0, 2)
    oseq, o_out, h_out, c_out = _tc_call(
        obs_p, act_t, ids_last, g.reshape(_B, _N, _P),
        W_gd, Wg_o, b_gnn, W_ih_oth, W_hh_oth, b_oth,
        W_ih_ego, W_hh_ego, b_ego)

    return oseq, (o_out, h_out, c_out)
